# Initial kernel scaffold; baseline (speedup 1.0000x reference)
import jax
import jax.numpy as jnp
from jax.experimental import pallas as pl


def kernel(x, pe):
    raise NotImplementedError("write your pallas kernel here")


# speedup vs baseline: 1.9141x; 1.9141x over previous
import jax
import jax.numpy as jnp
from jax.experimental import pallas as pl


_BL = 512


def _body(x_ref, pe_ref, o_ref):
    o_ref[...] = x_ref[...] + pe_ref[...]


def kernel(x, pe):
    b, l, d = x.shape
    nl = l // _BL
    return pl.pallas_call(
        _body,
        grid=(nl, b),
        in_specs=[
            pl.BlockSpec((1, _BL, d), lambda i, j: (j, i, 0)),
            pl.BlockSpec((_BL, d), lambda i, j: (i, 0)),
        ],
        out_specs=pl.BlockSpec((1, _BL, d), lambda i, j: (j, i, 0)),
        out_shape=jax.ShapeDtypeStruct((b, l, d), x.dtype),
    )(x, pe)
